# Initial kernel scaffold; baseline (speedup 1.0000x reference)
import jax
import jax.numpy as jnp
from jax.experimental import pallas as pl


def kernel(scores, target):
    raise NotImplementedError("write your pallas kernel here")


# speedup vs baseline: 3.1747x; 3.1747x over previous
import functools

import jax
import jax.numpy as jnp
from jax import lax
from jax.experimental import pallas as pl
from jax.experimental.pallas import tpu as pltpu
from jax.experimental.pallas import tpu_sc as plsc

BATCH = 1024
BRANCH = 16
NUM_INTERNAL = 273
NUM_EDGES = NUM_INTERNAL * BRANCH
L = 16
NC, NS = 2, 16
NW = NC * NS
BPW = BATCH // NW
ROWS_PW = 3 * BPW


def _sc_gather(scores_flat, target):
    mesh = plsc.VectorSubcoreMesh(core_axis_name="c", subcore_axis_name="s")

    @functools.partial(
        pl.kernel,
        mesh=mesh,
        compiler_params=pltpu.CompilerParams(use_tc_tiling_on_sc=False),
        out_type=(
            jax.ShapeDtypeStruct((NW, ROWS_PW, L), jnp.float32),
            jax.ShapeDtypeStruct((NW, ROWS_PW), jnp.int32),
        ),
        scratch_types=[
            pltpu.VMEM((BPW,), jnp.int32),
            pltpu.VMEM((ROWS_PW,), jnp.int32),
            pltpu.VMEM((ROWS_PW, L), jnp.float32),
            pltpu.VMEM((ROWS_PW,), jnp.int32),
            pltpu.SemaphoreType.DMA,
        ],
    )
    def k(scores_hbm, tgt_hbm, groups_hbm, pos_hbm, tgt_v, idx_v, rows_v,
          pos_v, sem):
        wid = lax.axis_index("s") * NC + lax.axis_index("c")
        base = wid * BPW
        pltpu.sync_copy(tgt_hbm.at[pl.ds(base, BPW)], tgt_v)
        lanes = lax.iota(jnp.int32, L)
        for i in range(BPW // L):
            t = tgt_v[pl.ds(i * L, L)]
            b_abs = base + i * L + lanes
            row_base = b_abs * NUM_INTERNAL
            idx_v[pl.ds(0 * BPW + i * L, L)] = row_base + 17 + (t >> 4)
            idx_v[pl.ds(1 * BPW + i * L, L)] = row_base + 1 + (t >> 8)
            idx_v[pl.ds(2 * BPW + i * L, L)] = row_base
            pos_v[pl.ds(0 * BPW + i * L, L)] = t & 15
            pos_v[pl.ds(1 * BPW + i * L, L)] = (t >> 4) & 15
            pos_v[pl.ds(2 * BPW + i * L, L)] = t >> 8
        pltpu.async_copy(scores_hbm.at[idx_v], rows_v, sem).wait()
        pltpu.sync_copy(rows_v, groups_hbm.at[wid])
        pltpu.sync_copy(pos_v, pos_hbm.at[wid])

    return k(scores_flat, target)


def _tc_body(g_ref, pos_ref, o_ref):
    g = g_ref[...]
    m = jnp.max(g, axis=-1, keepdims=True)
    s = jnp.sum(jnp.exp(g - m), axis=-1, keepdims=True)
    lse_sum = jnp.sum(jnp.log(s) + m)
    lane = lax.broadcasted_iota(jnp.int32, (NW * ROWS_PW, L), 1)
    picked = jnp.where(lane == pos_ref[...], g, 0.0)
    o_ref[0, 0] = (lse_sum - jnp.sum(picked)) * (1.0 / BATCH)


def kernel(scores, target):
    scores_flat = scores.reshape(BATCH * NUM_INTERNAL, L)
    groups, pos = _sc_gather(scores_flat, target)
    out = pl.pallas_call(
        _tc_body,
        out_shape=jax.ShapeDtypeStruct((1, 1), jnp.float32),
        in_specs=[
            pl.BlockSpec(memory_space=pltpu.VMEM),
            pl.BlockSpec(memory_space=pltpu.VMEM),
        ],
        out_specs=pl.BlockSpec(memory_space=pltpu.SMEM),
    )(groups.reshape(NW * ROWS_PW, L), pos.reshape(NW * ROWS_PW, 1))
    return out[0, 0]
